# Initial kernel scaffold; baseline (speedup 1.0000x reference)
import jax
import jax.numpy as jnp
from jax.experimental import pallas as pl


def kernel(ids, table):
    raise NotImplementedError("write your pallas kernel here")


# speedup vs baseline: 5.1066x; 5.1066x over previous
import functools

import jax
import jax.numpy as jnp
from jax import lax
from jax.experimental import pallas as pl
from jax.experimental.pallas import tpu as pltpu
from jax.experimental.pallas import tpu_sc as plsc

_B = 1024
_S = 200
_DIM = 128
_HALF = _DIM // 2
_BASE = 10000

_NC = 2
_NS = 16
_NW = _NC * _NS
_BPW = _B // _NW
_GCHUNK = _S // 2


def _sincos():
    inv_freq = 1.0 / (_BASE ** (jnp.arange(0, _HALF, dtype=jnp.float32) / _HALF))
    angles = jnp.arange(_S, dtype=jnp.float32)[:, None] * inv_freq[None, :]
    return jnp.sin(angles), jnp.cos(angles)


def _body(ids_ref, table_ref, sin_ref, cos_ref, out_ref,
          idx_v, rows_v, sin_v, cos_v, sem):
    wid = lax.axis_index("s") * _NC + lax.axis_index("c")

    pltpu.sync_copy(sin_ref, sin_v)
    pltpu.sync_copy(cos_ref, cos_v)

    def batch_body(k, carry):
        b = wid * _BPW + k
        pltpu.sync_copy(ids_ref.at[b], idx_v)
        cp0 = pltpu.async_copy(table_ref.at[idx_v.at[0]],
                               rows_v.at[pl.ds(0, _GCHUNK)], sem)
        cp1 = pltpu.async_copy(table_ref.at[idx_v.at[1]],
                               rows_v.at[pl.ds(_GCHUNK, _GCHUNK)], sem)
        cp0.wait()
        cp1.wait()

        def row_body(i, c2):
            for j in range(_HALF // 16):
                lo = pl.ds(j * 16, 16)
                hi = pl.ds(_HALF + j * 16, 16)
                t1 = rows_v[i, lo]
                t2 = rows_v[i, hi]
                cosv = cos_v[i, lo]
                sinv = sin_v[i, lo]
                rows_v[i, lo] = t1 * cosv - t2 * sinv
                rows_v[i, hi] = t1 * sinv + t2 * cosv
            return c2

        lax.fori_loop(0, _S, row_body, 0, unroll=False)
        pltpu.sync_copy(rows_v, out_ref.at[b])
        return carry

    lax.fori_loop(0, _BPW, batch_body, 0, unroll=False)


@jax.jit
def _run(ids2, table, sin, cos):
    mesh = plsc.VectorSubcoreMesh(core_axis_name="c", subcore_axis_name="s",
                                  num_cores=_NC, num_subcores=_NS)
    f = pl.kernel(
        _body,
        out_type=jax.ShapeDtypeStruct((_B, _S, _DIM), jnp.float32),
        mesh=mesh,
        scratch_types=[
            pltpu.VMEM((2, _GCHUNK), jnp.int32),
            pltpu.VMEM((_S, _DIM), jnp.float32),
            pltpu.VMEM((_S, _HALF), jnp.float32),
            pltpu.VMEM((_S, _HALF), jnp.float32),
            pltpu.SemaphoreType.DMA,
        ],
    )
    return f(ids2, table, sin, cos)


def kernel(ids, table):
    sin, cos = _sincos()
    ids2 = ids.reshape(_B, 2, _GCHUNK)
    return _run(ids2, table, sin, cos)
